# baseline (speedup 1.0000x reference)
import jax
import jax.numpy as jnp
from jax import lax
from jax.experimental import pallas as pl
from jax.experimental.pallas import tpu as pltpu
from jax.experimental.pallas import tpu_sc as plsc

N_NODES = 10000
N_EDGES = 320000
D = 128
NUM_GRAPHS = 64
NUM_CLASSES = 5

NC = 2
NS = 16
H = D // NC
LANES = 16
CH = 128
BLK = 18
NBLK = 9
NCHUNK = BLK * NBLK
E_PAD = NS * NCHUNK * CH
RPT = N_NODES // NS
RC = 125
NRC = RPT // RC
BIROWS = 8


def _when(cond, fn):
    if isinstance(cond, bool):
        if cond:
            fn()
    else:
        pl.when(cond)(fn)


def _sc_body(xe, srci, dsti, batchi, w12, ones_h, zrow_h, zdeg_h,
             pooled_out,
             xcache, agg, deg, pooled_sh,
             sidx, didx, rows_v, ones_v, db_v, bi_v, wv_v,
             gsem0, gsem1, gsem2, ssem0, ssem1, ssem2, isem0, isem1, dsem):
    c = lax.axis_index("c")
    s = lax.axis_index("s")
    nbase = s * RPT
    ibase = s * NCHUNK

    pltpu.sync_copy(xe.at[pl.ds(c * N_NODES + nbase, RPT)],
                    xcache.at[pl.ds(nbase, RPT)])
    pltpu.sync_copy(zrow_h, agg.at[pl.ds(nbase, RPT)])
    pltpu.sync_copy(zdeg_h, deg.at[pl.ds(nbase, RPT)])

    @pl.when(s == 0)
    def _():
        pltpu.sync_copy(zrow_h.at[pl.ds(0, NUM_GRAPHS)], pooled_sh)

    pltpu.sync_copy(ones_h, ones_v)
    pltpu.sync_copy(w12, wv_v)
    pltpu.sync_copy(batchi.at[pl.ds(s * BIROWS, BIROWS)], bi_v)
    plsc.subcore_barrier()

    gsems = (gsem0, gsem1, gsem2)
    ssems = (ssem0, ssem1, ssem2)

    def edge_pass(with_deg):

        def run_block(m, half, first_block):
            nxt = 1 - half

            def _prefetch():
                off = ibase + (m + 1) * BLK
                pltpu.async_copy(srci.at[pl.ds(off, BLK)], sidx.at[nxt],
                                 isem0)
                pltpu.async_copy(dsti.at[pl.ds(off, BLK)], didx.at[nxt],
                                 isem1)

            _when(m + 1 < NBLK, _prefetch)

            for q in range(BLK):
                b = q % 3
                pltpu.make_async_copy(xcache.at[sidx.at[half].at[q]],
                                      rows_v.at[b], gsems[b]).wait()
                pltpu.async_copy(rows_v.at[b], agg.at[didx.at[half].at[q]],
                                 ssems[b], add=True)
                if with_deg:
                    pltpu.async_copy(ones_v, deg.at[didx.at[half].at[q]],
                                     dsem, add=True)
                    if not (first_block and q < 3):
                        pltpu.make_async_copy(ones_v,
                                              deg.at[didx.at[half].at[q]],
                                              dsem).wait()
                if not (first_block and q == 0):
                    pb = (q + 2) % 3
                    pltpu.make_async_copy(rows_v.at[pb],
                                          agg.at[didx.at[half].at[q]],
                                          ssems[pb]).wait()
                if q == BLK - 3:
                    def _pwait():
                        off = ibase + (m + 1) * BLK
                        pltpu.make_async_copy(srci.at[pl.ds(off, BLK)],
                                              sidx.at[nxt], isem0).wait()
                        pltpu.make_async_copy(dsti.at[pl.ds(off, BLK)],
                                              didx.at[nxt], isem1).wait()

                    _when(m + 1 < NBLK, _pwait)
                gb = (q + 2) % 3
                if q < BLK - 2:
                    pltpu.async_copy(xcache.at[sidx.at[half].at[q + 2]],
                                     rows_v.at[gb], gsems[gb])
                else:
                    def _gfire():
                        pltpu.async_copy(
                            xcache.at[sidx.at[nxt].at[q - (BLK - 2)]],
                            rows_v.at[gb], gsems[gb])

                    _when(m + 1 < NBLK, _gfire)

        pltpu.sync_copy(srci.at[pl.ds(ibase, BLK)], sidx.at[0])
        pltpu.sync_copy(dsti.at[pl.ds(ibase, BLK)], didx.at[0])
        pltpu.async_copy(xcache.at[sidx.at[0].at[0]], rows_v.at[0], gsem0)
        pltpu.async_copy(xcache.at[sidx.at[0].at[1]], rows_v.at[1], gsem1)

        run_block(0, 0, True)

        def superblock(g, carry):
            run_block(2 * g + 1, 1, False)
            run_block(2 * g + 2, 0, False)
            return carry

        lax.fori_loop(0, (NBLK - 1) // 2, superblock, 0)

        fb = (NCHUNK - 1) % 3
        pltpu.make_async_copy(rows_v.at[fb], agg.at[didx.at[0].at[BLK - 1]],
                              ssems[fb]).wait()
        if with_deg:
            for _ in range(3):
                pltpu.make_async_copy(ones_v, deg.at[didx.at[0].at[0]],
                                      dsem).wait()

    def update_pass(w_row, last):
        wv = wv_v[w_row, :]
        for k in range(NRC):
            rb = nbase + k * RC
            xb = rows_v.at[0].at[pl.ds(0, RC)]
            ab = rows_v.at[1].at[pl.ds(0, RC)]
            pltpu.sync_copy(xcache.at[pl.ds(rb, RC)], xb)
            pltpu.sync_copy(agg.at[pl.ds(rb, RC)], ab)
            pltpu.sync_copy(deg.at[pl.ds(rb, RC)], db_v)

            def row(r, carry):
                dvec = db_v[r, :]
                for j in range(H // LANES):
                    xv = rows_v[0, r, pl.ds(LANES * j, LANES)]
                    av = rows_v[1, r, pl.ds(LANES * j, LANES)]
                    rows_v[0, r, pl.ds(LANES * j, LANES)] = (
                        xv + wv * (av - dvec * xv))
                return carry

            lax.fori_loop(0, RC, row, 0, unroll=2)
            if not last:
                pltpu.sync_copy(xb, xcache.at[pl.ds(rb, RC)])
            else:
                pltpu.sync_copy(xb, pooled_sh.at[bi_v.at[k]], add=True)
        if not last:
            pltpu.sync_copy(zrow_h, agg.at[pl.ds(nbase, RPT)])

    edge_pass(True)
    plsc.subcore_barrier()
    update_pass(0, False)
    plsc.subcore_barrier()
    edge_pass(False)
    plsc.subcore_barrier()
    update_pass(1, True)
    plsc.subcore_barrier()

    @pl.when(s == 0)
    def _():
        pltpu.sync_copy(pooled_sh,
                        pooled_out.at[pl.ds(c * NUM_GRAPHS, NUM_GRAPHS)])


def _run_sc(xe, srci, dsti, batchi, w12, ones_h, zrow_h, zdeg_h):
    mesh = plsc.VectorSubcoreMesh(core_axis_name="c", subcore_axis_name="s",
                                  num_cores=NC, num_subcores=NS)
    f = pl.kernel(
        _sc_body,
        out_type=jax.ShapeDtypeStruct((NC * NUM_GRAPHS, H), jnp.float32),
        mesh=mesh,
        compiler_params=pltpu.CompilerParams(use_tc_tiling_on_sc=False),
        scratch_types=[
            pltpu.VMEM_SHARED((N_NODES, H), jnp.float32),
            pltpu.VMEM_SHARED((N_NODES, H), jnp.float32),
            pltpu.VMEM_SHARED((N_NODES, LANES), jnp.float32),
            pltpu.VMEM_SHARED((NUM_GRAPHS, H), jnp.float32),
            pltpu.VMEM((2, BLK, CH), jnp.int32),
            pltpu.VMEM((2, BLK, CH), jnp.int32),
            pltpu.VMEM((3, CH, H), jnp.float32),
            pltpu.VMEM((CH, LANES), jnp.float32),
            pltpu.VMEM((RC, LANES), jnp.float32),
            pltpu.VMEM((BIROWS, RC), jnp.int32),
            pltpu.VMEM((2, LANES), jnp.float32),
            pltpu.SemaphoreType.DMA,
            pltpu.SemaphoreType.DMA,
            pltpu.SemaphoreType.DMA,
            pltpu.SemaphoreType.DMA,
            pltpu.SemaphoreType.DMA,
            pltpu.SemaphoreType.DMA,
            pltpu.SemaphoreType.DMA,
            pltpu.SemaphoreType.DMA,
            pltpu.SemaphoreType.DMA,
        ],
    )
    return f(xe, srci, dsti, batchi, w12, ones_h, zrow_h, zdeg_h)


def _mm_body(p_ref, w_ref, b_ref, o_ref):
    o_ref[...] = (
        jnp.dot(p_ref[...], w_ref[...], preferred_element_type=jnp.float32)
        + b_ref[...]
    )


def _linear(pooled, lin_w, lin_b):
    return pl.pallas_call(
        _mm_body,
        out_shape=jax.ShapeDtypeStruct((NUM_GRAPHS, NUM_CLASSES), jnp.float32),
    )(pooled, lin_w, lin_b)


def kernel(x, edge_index, batch, w1, w2, lin_w, lin_b):
    ei = edge_index.astype(jnp.int32)
    pad = E_PAD - N_EDGES
    src = jnp.concatenate([ei[0], jnp.zeros((pad,), jnp.int32)])
    dst = jnp.concatenate([ei[1], jnp.zeros((pad,), jnp.int32)])
    srci = src.reshape(NS * NCHUNK, CH)
    dsti = dst.reshape(NS * NCHUNK, CH)
    b3 = batch.astype(jnp.int32).reshape(NS, NRC, RC)
    b3 = jnp.concatenate(
        [b3, jnp.zeros((NS, BIROWS - NRC, RC), jnp.int32)], axis=1)
    batchi = b3.reshape(NS * BIROWS, RC)
    xe = jnp.concatenate([x[:, :H], x[:, H:]], axis=0)
    w12 = jnp.stack([jnp.full((LANES,), w1, jnp.float32),
                     jnp.full((LANES,), w2, jnp.float32)])
    ones_h = jnp.ones((CH, LANES), jnp.float32)
    zrow_h = jnp.zeros((RPT, H), jnp.float32)
    zdeg_h = jnp.zeros((RPT, LANES), jnp.float32)
    pooled2 = _run_sc(xe, srci, dsti, batchi, w12, ones_h, zrow_h, zdeg_h)
    pooled = jnp.concatenate([pooled2[:NUM_GRAPHS], pooled2[NUM_GRAPHS:]],
                             axis=1)
    return _linear(pooled, lin_w, lin_b.reshape(1, NUM_CLASSES))
